# baseline (speedup 1.0000x reference)
import functools
import jax
import jax.numpy as jnp
from jax import lax
from jax.experimental import pallas as pl
from jax.experimental.pallas import tpu as pltpu
from jax.experimental.pallas import tpu_sc as plsc

B, N, D = 4, 8192, 768
NC, NS, L = 2, 16, 16
NW = NC * NS
PPW = N // NW
R = 16
RD = R * D
NCH = PPW // R
T = NCH * B
NB = 4


def _sc_add(inp2, tab1):
    mesh = plsc.VectorSubcoreMesh(core_axis_name="c", subcore_axis_name="s")

    @functools.partial(
        pl.kernel,
        out_type=jax.ShapeDtypeStruct((B, N * D), jnp.float32),
        mesh=mesh,
        scratch_types=[
            pltpu.VMEM((2, RD), jnp.float32),
            pltpu.VMEM((NB, RD), jnp.float32),
            pltpu.SemaphoreType.DMA((2,)),
            pltpu.SemaphoreType.DMA((NB,)),
            pltpu.SemaphoreType.DMA((NB,)),
        ],
    )
    def k(inp_hbm, tab_hbm, out_hbm, tbuf, ibuf, tsem, lsem, ssem):
        wid = lax.axis_index("s") * NC + lax.axis_index("c")
        w_base = wid * PPW * D

        def start_load(it):
            c = it // B
            b = it % B
            s = it % NB
            pltpu.async_copy(
                inp_hbm.at[b, pl.ds(w_base + c * RD, RD)], ibuf.at[s], lsem.at[s])

        def wait_load(s):
            pltpu.make_async_copy(
                inp_hbm.at[0, pl.ds(0, RD)], ibuf.at[s], lsem.at[s]).wait()

        def start_store(it):
            c = it // B
            b = it % B
            s = it % NB
            pltpu.async_copy(
                ibuf.at[s], out_hbm.at[b, pl.ds(w_base + c * RD, RD)], ssem.at[s])

        def wait_store(s):
            pltpu.make_async_copy(
                ibuf.at[s], out_hbm.at[0, pl.ds(0, RD)], ssem.at[s]).wait()

        def start_tload(c):
            pltpu.async_copy(
                tab_hbm.at[pl.ds(w_base + c * RD, RD)], tbuf.at[c % 2], tsem.at[c % 2])

        def wait_tload(c):
            pltpu.make_async_copy(
                tab_hbm.at[pl.ds(0, RD)], tbuf.at[c % 2], tsem.at[c % 2]).wait()

        start_tload(0)
        start_load(0)
        start_load(1)

        def body(it, carry):
            c = it // B
            b = it % B
            s = it % NB
            tk = c % 2

            @pl.when(it + 2 < T)
            def _():
                @pl.when(it + 2 >= NB)
                def _():
                    wait_store((it + 2) % NB)
                start_load(it + 2)

            @pl.when(b == 0)
            def _():
                wait_tload(c)

                @pl.when(c + 1 < NCH)
                def _():
                    start_tload(c + 1)

            wait_load(s)

            tb = tbuf.at[tk]
            ib = ibuf.at[s]

            @plsc.parallel_loop(0, RD, step=L, unroll=8)
            def add_vec(i):
                plsc.addupdate(ib.at[pl.ds(i, L)], tb[pl.ds(i, L)])

            start_store(it)
            return carry

        lax.fori_loop(0, T, body, 0)
        for s in range(NB):
            wait_store(s)

    return k(inp2, tab1)


def kernel(inputs, pos_table):
    out2 = _sc_add(inputs.reshape(B, N * D), pos_table.reshape(N * D))
    return out2.reshape(B, N, D)


# speedup vs baseline: 2.9667x; 2.9667x over previous
import functools
import jax
import jax.numpy as jnp
from jax import lax
from jax.experimental import pallas as pl
from jax.experimental.pallas import tpu as pltpu
from jax.experimental.pallas import tpu_sc as plsc

B, N, D = 4, 8192, 768
NC, NS, L = 2, 16, 16
NW = NC * NS
PPW = N // NW
R = 16
NCH = PPW // R
NV = D // L
T = NCH * B
NB = 4


def _sc_add(inputs, pos_table):
    mesh = plsc.VectorSubcoreMesh(core_axis_name="c", subcore_axis_name="s")

    @functools.partial(
        pl.kernel,
        out_type=jax.ShapeDtypeStruct((B, N, D), jnp.float32),
        mesh=mesh,
        scratch_types=[
            pltpu.VMEM((2, R, D), jnp.float32),
            pltpu.VMEM((NB, R, D), jnp.float32),
            pltpu.SemaphoreType.DMA((2,)),
            pltpu.SemaphoreType.DMA((NB,)),
            pltpu.SemaphoreType.DMA((NB,)),
        ],
    )
    def k(inp_hbm, tab_hbm, out_hbm, tbuf, ibuf, tsem, lsem, ssem):
        wid = lax.axis_index("s") * NC + lax.axis_index("c")
        p_base = wid * PPW

        def start_load(c, b, s):
            pltpu.async_copy(
                inp_hbm.at[b, pl.ds(p_base + c * R, R)], ibuf.at[s], lsem.at[s])

        def wait_load(s):
            pltpu.make_async_copy(
                inp_hbm.at[0, pl.ds(0, R)], ibuf.at[s], lsem.at[s]).wait()

        def start_store(c, b, s):
            pltpu.async_copy(
                ibuf.at[s], out_hbm.at[b, pl.ds(p_base + c * R, R)], ssem.at[s])

        def wait_store(s):
            pltpu.make_async_copy(
                ibuf.at[s], out_hbm.at[0, pl.ds(0, R)], ssem.at[s]).wait()

        def start_tload(c, tk):
            pltpu.async_copy(
                tab_hbm.at[pl.ds(p_base + c * R, R)], tbuf.at[tk], tsem.at[tk])

        def wait_tload(tk):
            pltpu.make_async_copy(
                tab_hbm.at[pl.ds(0, R)], tbuf.at[0], tsem.at[tk]).wait()

        start_tload(0, 0)
        start_load(0, 0, 0)
        start_load(0, 1, 1)

        def round_(g, carry):
            for cc in range(2):
                c = 2 * g + cc
                for b in range(B):
                    it = (2 * g + cc) * B + b
                    s = b
                    s2 = (b + 2) % NB
                    c2 = c if b < 2 else c + 1
                    b2 = b + 2 if b < 2 else b - 2

                    @pl.when(it + 2 < T)
                    def _(c2=c2, b2=b2, s2=s2, it=it):
                        @pl.when(it + 2 >= NB)
                        def _():
                            wait_store(s2)
                        start_load(c2, b2, s2)

                    if b == 0:
                        wait_tload(cc)

                        @pl.when(c + 1 < NCH)
                        def _(c=c, cc=cc):
                            start_tload(c + 1, 1 - cc)

                    wait_load(s)
                    tb = tbuf.at[cc]
                    ib = ibuf.at[s]

                    def add_row(r, carry3, tb=tb, ib=ib):
                        for j in range(NV):
                            plsc.addupdate(
                                ib.at[r, pl.ds(j * L, L)], tb[r, pl.ds(j * L, L)])
                        return carry3

                    lax.fori_loop(0, R, add_row, 0)
                    start_store(c, b, s)
            return carry

        lax.fori_loop(0, NCH // 2, round_, 0)
        for s in range(NB):
            wait_store(s)

    return k(inputs, pos_table)


def kernel(inputs, pos_table):
    return _sc_add(inputs, pos_table)


# speedup vs baseline: 3.0569x; 1.0304x over previous
import functools
import jax
import jax.numpy as jnp
from jax import lax
from jax.experimental import pallas as pl
from jax.experimental.pallas import tpu as pltpu
from jax.experimental.pallas import tpu_sc as plsc

B, N, D = 4, 8192, 768
NC, NS, L = 2, 16, 16
NW = NC * NS
PPW = N // NW
R = 16
NCH = PPW // R
NV = D // L
T = NCH * B
NB = 4


def _sc_add(inputs, pos_table):
    mesh = plsc.VectorSubcoreMesh(core_axis_name="c", subcore_axis_name="s")

    @functools.partial(
        pl.kernel,
        out_type=jax.ShapeDtypeStruct((B, N, D), jnp.float32),
        mesh=mesh,
        scratch_types=[
            pltpu.VMEM((2, R, D), jnp.float32),
            pltpu.VMEM((NB, R, D), jnp.float32),
            pltpu.SemaphoreType.DMA((2,)),
            pltpu.SemaphoreType.DMA((NB,)),
            pltpu.SemaphoreType.DMA((NB,)),
        ],
    )
    def k(inp_hbm, tab_hbm, out_hbm, tbuf, ibuf, tsem, lsem, ssem):
        wid = lax.axis_index("s") * NC + lax.axis_index("c")
        p_base = wid * PPW

        def start_load(c, b, s):
            pltpu.async_copy(
                inp_hbm.at[b, pl.ds(p_base + c * R, R)], ibuf.at[s], lsem.at[s])

        def wait_load(s):
            pltpu.make_async_copy(
                inp_hbm.at[0, pl.ds(0, R)], ibuf.at[s], lsem.at[s]).wait()

        def start_store(c, b, s):
            pltpu.async_copy(
                ibuf.at[s], out_hbm.at[b, pl.ds(p_base + c * R, R)], ssem.at[s])

        def wait_store(s):
            pltpu.make_async_copy(
                ibuf.at[s], out_hbm.at[0, pl.ds(0, R)], ssem.at[s]).wait()

        def start_tload(c, tk):
            pltpu.async_copy(
                tab_hbm.at[pl.ds(p_base + c * R, R)], tbuf.at[tk], tsem.at[tk])

        def wait_tload(tk):
            pltpu.make_async_copy(
                tab_hbm.at[pl.ds(0, R)], tbuf.at[0], tsem.at[tk]).wait()

        start_tload(0, 0)
        start_load(0, 0, 0)
        start_load(0, 1, 1)

        def round_(g, carry):
            for cc in range(2):
                c = 2 * g + cc
                for b in range(B):
                    it = (2 * g + cc) * B + b
                    s = b
                    s2 = (b + 2) % NB
                    c2 = c if b < 2 else c + 1
                    b2 = b + 2 if b < 2 else b - 2

                    @pl.when(it + 2 < T)
                    def _(c2=c2, b2=b2, s2=s2, it=it):
                        @pl.when(it + 2 >= NB)
                        def _():
                            wait_store(s2)
                        start_load(c2, b2, s2)

                    if b == 0:
                        wait_tload(cc)

                        @pl.when(c + 1 < NCH)
                        def _(c=c, cc=cc):
                            start_tload(c + 1, 1 - cc)

                    wait_load(s)
                    tb = tbuf.at[cc]
                    ib = ibuf.at[s]

                    def add_row(r2, carry3, tb=tb, ib=ib):
                        for rr in range(2):
                            r = 2 * r2 + rr
                            for j in range(NV):
                                plsc.addupdate(
                                    ib.at[r, pl.ds(j * L, L)], tb[r, pl.ds(j * L, L)])
                        return carry3

                    lax.fori_loop(0, R // 2, add_row, 0)
                    start_store(c, b, s)
            return carry

        lax.fori_loop(0, NCH // 2, round_, 0)
        for s in range(NB):
            wait_store(s)

    return k(inputs, pos_table)


def kernel(inputs, pos_table):
    return _sc_add(inputs, pos_table)
